# Initial kernel scaffold; baseline (speedup 1.0000x reference)
import jax
import jax.numpy as jnp
from jax.experimental import pallas as pl


def kernel(train_view, k, edge_inx_list, update_epoch, alpha_list, Win, b_in, W0, b0, W1, b1, W2, b2):
    raise NotImplementedError("write your pallas kernel here")


# speedup vs baseline: 2.1862x; 2.1862x over previous
import functools

import jax
import jax.numpy as jnp
from jax import lax
from jax.experimental import pallas as pl
from jax.experimental.pallas import tpu as pltpu
from jax.experimental.pallas import tpu_sc as plsc

F32 = jnp.float32

BN1 = 512
BN3 = 512
C = 32


def _k1_body(tv_ref, win_ref, bin_ref, w0_ref, b0_ref, w1_ref, b1_ref,
             w2_ref, b2_ref, w1h_ref, pre_ref):
    V = tv_ref.shape[0]
    tv = tv_ref[...]
    w0 = w0_ref[...]
    w1 = w1_ref[...]
    w2 = w2_ref[...]
    b0 = b0_ref[...]
    b1 = b1_ref[...]
    b2 = b2_ref[...]
    hi = [jnp.dot(tv[x], win_ref[x], preferred_element_type=F32) + bin_ref[x]
          for x in range(V)]
    s = hi[0]
    for x in range(1, V):
        s = s + hi[x]
    inv = 1.0 / (V - 1)
    for x in range(V):
        hb = jnp.dot(hi[x], w0, preferred_element_type=F32) + b0
        w1h_ref[x] = jnp.dot(hb, w1, preferred_element_type=F32) + b1
        pre_ref[x] = hb + jnp.dot(s - hi[x], w2, preferred_element_type=F32) * inv + b2


def _k3_body(pre_ref, nbr_ref, scale_ref, out_ref):
    V = pre_ref.shape[0]
    acc = jnp.maximum(pre_ref[0] + nbr_ref[0], 0.0)
    for x in range(1, V):
        acc = acc + jnp.maximum(pre_ref[x] + nbr_ref[x], 0.0)
    out_ref[...] = acc * scale_ref[...]


def _make_sc_gather(n_rows, n_pad_per_view, stride, hid, n_view):
    info = plsc.get_sparse_core_info()
    nc, ns, nl = info.num_cores, info.num_subcores, info.num_lanes
    nw = nc * ns
    nt = n_view * n_pad_per_view
    per_w = nt // nw
    n_chunk = per_w // C
    assert per_w % C == 0 and nt % nw == 0
    lane_groups = hid // nl

    mesh = plsc.VectorSubcoreMesh(core_axis_name="c", subcore_axis_name="s")

    @functools.partial(
        pl.kernel,
        mesh=mesh,
        out_type=jax.ShapeDtypeStruct((nt, hid), F32),
        scratch_types=[
            pltpu.VMEM((stride * C,), jnp.int32),
            pltpu.VMEM((stride, C, hid), F32),
            pltpu.VMEM((C, hid), F32),
            pltpu.VMEM((stride * nl,), F32),
            pltpu.SemaphoreType.DMA,
        ],
    )
    def sc_gather(table_hbm, idx_hbm, w_hbm, nbr_hbm, idx_v, g_v, out_v, w_v, sem):
        wid = lax.axis_index("s") * nc + lax.axis_index("c")
        base0 = wid * per_w
        pltpu.sync_copy(w_hbm, w_v)
        wvecs = [w_v[pl.ds(j * nl, nl)] for j in range(stride)]

        def chunk_body(ci, carry):
            base = base0 + ci * C
            pltpu.sync_copy(idx_hbm.at[pl.ds(base * stride, stride * C)], idx_v)
            off = (base // n_pad_per_view) * n_rows
            for t in range(stride * C // nl):
                sl = pl.ds(t * nl, nl)
                idx_v[sl] = idx_v[sl] + off
            copies = [
                pltpu.async_copy(table_hbm.at[idx_v.at[pl.ds(j * C, C)]],
                                 g_v.at[j], sem)
                for j in range(stride)
            ]
            for cp in copies:
                cp.wait()

            def row_body(r, carry2):
                def lane_body(li, carry3):
                    sl = pl.ds(li * nl, nl)
                    acc = g_v[0, r, sl] * wvecs[0]
                    for j in range(1, stride):
                        acc = acc + g_v[j, r, sl] * wvecs[j]
                    out_v[r, sl] = acc
                    return carry3
                return lax.fori_loop(0, lane_groups, lane_body, carry2)

            lax.fori_loop(0, C, row_body, 0)
            pltpu.sync_copy(out_v, nbr_hbm.at[pl.ds(base, C)])
            return carry

        lax.fori_loop(0, n_chunk, chunk_body, 0)

    return sc_gather


def kernel(train_view, k, edge_inx_list, update_epoch, alpha_list, Win, b_in,
           W0, b0, W1, b1, W2, b2):
    V, N, D = train_view.shape
    HID = W0.shape[0]
    STRIDE = edge_inx_list.shape[2] // N

    grid1 = pl.cdiv(N, BN1)
    w1h, pre = pl.pallas_call(
        _k1_body,
        grid=(grid1,),
        in_specs=[
            pl.BlockSpec((V, BN1, D), lambda i: (0, i, 0)),
            pl.BlockSpec((V, D, HID), lambda i: (0, 0, 0)),
            pl.BlockSpec((V, 1, HID), lambda i: (0, 0, 0)),
            pl.BlockSpec((HID, HID), lambda i: (0, 0)),
            pl.BlockSpec((1, HID), lambda i: (0, 0)),
            pl.BlockSpec((HID, HID), lambda i: (0, 0)),
            pl.BlockSpec((1, HID), lambda i: (0, 0)),
            pl.BlockSpec((HID, HID), lambda i: (0, 0)),
            pl.BlockSpec((1, HID), lambda i: (0, 0)),
        ],
        out_specs=[
            pl.BlockSpec((V, BN1, HID), lambda i: (0, i, 0)),
            pl.BlockSpec((V, BN1, HID), lambda i: (0, i, 0)),
        ],
        out_shape=[
            jax.ShapeDtypeStruct((V, N, HID), F32),
            jax.ShapeDtypeStruct((V, N, HID), F32),
        ],
    )(train_view, Win, b_in.reshape(V, 1, HID), W0, b0.reshape(1, HID),
      W1, b1.reshape(1, HID), W2, b2.reshape(1, HID))

    table = w1h.reshape(V * N, HID)

    info = plsc.get_sparse_core_info()
    nw = info.num_cores * info.num_subcores
    unit = nw * C
    NPAD = pl.cdiv(N, unit) * unit
    nb = edge_inx_list[:, 1, :].reshape(V, N, STRIDE).astype(jnp.int32)
    nbT = jnp.transpose(nb, (2, 0, 1))
    idx_flat = jnp.pad(nbT, ((0, 0), (0, 0), (0, NPAD - N))).reshape(
        STRIDE, V * NPAD)
    n_chunk = (V * NPAD) // (nw * C)
    idx_chunks = idx_flat.reshape(STRIDE, nw, n_chunk, C).transpose(
        1, 2, 0, 3).reshape(-1)

    kf = jnp.asarray(k, F32)
    wslot = jnp.where(jnp.arange(STRIDE) < k,
                      1.0 / jnp.maximum(kf, 1.0), 0.0).astype(F32)
    wmat = jnp.broadcast_to(wslot[:, None], (STRIDE, 16)).reshape(-1)

    sc = _make_sc_gather(N, NPAD, STRIDE, HID, V)
    nbr = sc(table, idx_chunks, wmat).reshape(V, NPAD, HID)

    gate = (jnp.asarray(update_epoch) > 0).astype(F32) / V
    scale_row = jnp.broadcast_to(gate, (1, HID)).astype(F32)
    grid3 = pl.cdiv(N, BN3)
    out = pl.pallas_call(
        _k3_body,
        grid=(grid3,),
        in_specs=[
            pl.BlockSpec((V, BN3, HID), lambda i: (0, i, 0)),
            pl.BlockSpec((V, BN3, HID), lambda i: (0, i, 0)),
            pl.BlockSpec((1, HID), lambda i: (0, 0)),
        ],
        out_specs=pl.BlockSpec((BN3, HID), lambda i: (i, 0)),
        out_shape=jax.ShapeDtypeStruct((N, HID), F32),
    )(pre, nbr, scale_row)
    return out


# speedup vs baseline: 2.9520x; 1.3503x over previous
import functools

import jax
import jax.numpy as jnp
from jax import lax
from jax.experimental import pallas as pl
from jax.experimental.pallas import tpu as pltpu
from jax.experimental.pallas import tpu_sc as plsc

F32 = jnp.float32

BN1 = 512
BN3 = 512
C = 32


def _k1_body(tv_ref, win_ref, bin_ref, w0_ref, b0_ref, w1_ref, b1_ref,
             w2_ref, b2_ref, w1h_ref, pre_ref):
    V = tv_ref.shape[0]
    tv = tv_ref[...]
    w0 = w0_ref[...]
    w1 = w1_ref[...]
    w2 = w2_ref[...]
    b0 = b0_ref[...]
    b1 = b1_ref[...]
    b2 = b2_ref[...]
    hi = [jnp.dot(tv[x], win_ref[x], preferred_element_type=F32) + bin_ref[x]
          for x in range(V)]
    s = hi[0]
    for x in range(1, V):
        s = s + hi[x]
    inv = 1.0 / (V - 1)
    for x in range(V):
        hb = jnp.dot(hi[x], w0, preferred_element_type=F32) + b0
        w1h_ref[x] = jnp.dot(hb, w1, preferred_element_type=F32) + b1
        pre_ref[x] = hb + jnp.dot(s - hi[x], w2, preferred_element_type=F32) * inv + b2


def _k3_body(pre_ref, nbr_ref, scale_ref, out_ref):
    V = pre_ref.shape[0]
    acc = jnp.maximum(pre_ref[0] + nbr_ref[0], 0.0)
    for x in range(1, V):
        acc = acc + jnp.maximum(pre_ref[x] + nbr_ref[x], 0.0)
    out_ref[...] = acc * scale_ref[...]


def _make_sc_gather(n_rows, n_pad_per_view, stride, hid, n_view):
    info = plsc.get_sparse_core_info()
    nc, ns, nl = info.num_cores, info.num_subcores, info.num_lanes
    nw = nc * ns
    nt = n_view * n_pad_per_view
    per_w = nt // nw
    n_chunk = per_w // C
    assert per_w % C == 0 and nt % nw == 0
    lane_groups = hid // nl

    assert n_chunk % 2 == 0
    mesh = plsc.VectorSubcoreMesh(core_axis_name="c", subcore_axis_name="s")

    @functools.partial(
        pl.kernel,
        mesh=mesh,
        out_type=jax.ShapeDtypeStruct((nt, hid), F32),
        scratch_types=[
            pltpu.VMEM((stride * C,), jnp.int32),
            pltpu.VMEM((stride * C,), jnp.int32),
            pltpu.VMEM((stride, C, hid), F32),
            pltpu.VMEM((stride, C, hid), F32),
            pltpu.VMEM((C, hid), F32),
            pltpu.VMEM((C, hid), F32),
            pltpu.VMEM((stride * nl,), F32),
            pltpu.SemaphoreType.DMA,
            pltpu.SemaphoreType.DMA,
            pltpu.SemaphoreType.DMA,
            pltpu.SemaphoreType.DMA,
            pltpu.SemaphoreType.DMA,
            pltpu.SemaphoreType.DMA,
        ],
    )
    def sc_gather(table_hbm, idx_hbm, w_hbm, nbr_hbm,
                  idx_v0, idx_v1, g_v0, g_v1, out_v0, out_v1, w_v,
                  sem_i0, sem_i1, sem_g0, sem_g1, sem_o0, sem_o1):
        idx_b = (idx_v0, idx_v1)
        g_b = (g_v0, g_v1)
        out_b = (out_v0, out_v1)
        sem_i = (sem_i0, sem_i1)
        sem_g = (sem_g0, sem_g1)
        sem_o = (sem_o0, sem_o1)
        wid = lax.axis_index("s") * nc + lax.axis_index("c")
        base0 = wid * per_w
        pltpu.sync_copy(w_hbm, w_v)
        wvecs = [w_v[pl.ds(j * nl, nl)] for j in range(stride)]

        def idx_copy(ci, b):
            return pltpu.make_async_copy(
                idx_hbm.at[pl.ds((base0 + ci * C) * stride, stride * C)],
                idx_b[b], sem_i[b])

        def gather_copy(b, j):
            return pltpu.make_async_copy(
                table_hbm.at[idx_b[b].at[pl.ds(j * C, C)]], g_b[b].at[j],
                sem_g[b])

        def out_copy(ci, b):
            return pltpu.make_async_copy(
                out_b[b], nbr_hbm.at[pl.ds(base0 + ci * C, C)], sem_o[b])

        def fire_gathers(ci, b):
            off = ((base0 + ci * C) // n_pad_per_view) * n_rows
            for t in range(stride * C // nl):
                sl = pl.ds(t * nl, nl)
                idx_b[b][sl] = idx_b[b][sl] + off
            for j in range(stride):
                gather_copy(b, j).start()

        idx_copy(0, 0).start()
        idx_copy(0, 0).wait()
        fire_gathers(0, 0)
        idx_copy(1, 1).start()

        def chunk_step(ci, b):
            for j in range(stride):
                gather_copy(b, j).wait()

            @pl.when(ci + 1 < n_chunk)
            def _fire_next():
                idx_copy(ci + 1, 1 - b).wait()
                fire_gathers(ci + 1, 1 - b)

            @pl.when(ci + 2 < n_chunk)
            def _prefetch_idx():
                idx_copy(ci + 2, b).start()

            @pl.when(ci >= 2)
            def _drain_out():
                out_copy(ci - 2, b).wait()

            def row_body(r, carry2):
                def lane_body(li, carry3):
                    sl = pl.ds(li * nl, nl)
                    acc = g_b[b][0, r, sl] * wvecs[0]
                    for j in range(1, stride):
                        acc = acc + g_b[b][j, r, sl] * wvecs[j]
                    out_b[b][r, sl] = acc
                    return carry3
                return lax.fori_loop(0, lane_groups, lane_body, carry2)

            lax.fori_loop(0, C, row_body, 0)
            out_copy(ci, b).start()

        def group_body(gi, carry):
            for b in range(2):
                chunk_step(gi * 2 + b, b)
            return carry

        lax.fori_loop(0, n_chunk // 2, group_body, 0)
        out_copy(n_chunk - 2, 0).wait()
        out_copy(n_chunk - 1, 1).wait()

    return sc_gather


def kernel(train_view, k, edge_inx_list, update_epoch, alpha_list, Win, b_in,
           W0, b0, W1, b1, W2, b2):
    V, N, D = train_view.shape
    HID = W0.shape[0]
    STRIDE = edge_inx_list.shape[2] // N

    grid1 = pl.cdiv(N, BN1)
    w1h, pre = pl.pallas_call(
        _k1_body,
        grid=(grid1,),
        in_specs=[
            pl.BlockSpec((V, BN1, D), lambda i: (0, i, 0)),
            pl.BlockSpec((V, D, HID), lambda i: (0, 0, 0)),
            pl.BlockSpec((V, 1, HID), lambda i: (0, 0, 0)),
            pl.BlockSpec((HID, HID), lambda i: (0, 0)),
            pl.BlockSpec((1, HID), lambda i: (0, 0)),
            pl.BlockSpec((HID, HID), lambda i: (0, 0)),
            pl.BlockSpec((1, HID), lambda i: (0, 0)),
            pl.BlockSpec((HID, HID), lambda i: (0, 0)),
            pl.BlockSpec((1, HID), lambda i: (0, 0)),
        ],
        out_specs=[
            pl.BlockSpec((V, BN1, HID), lambda i: (0, i, 0)),
            pl.BlockSpec((V, BN1, HID), lambda i: (0, i, 0)),
        ],
        out_shape=[
            jax.ShapeDtypeStruct((V, N, HID), F32),
            jax.ShapeDtypeStruct((V, N, HID), F32),
        ],
    )(train_view, Win, b_in.reshape(V, 1, HID), W0, b0.reshape(1, HID),
      W1, b1.reshape(1, HID), W2, b2.reshape(1, HID))

    table = w1h.reshape(V * N, HID)

    info = plsc.get_sparse_core_info()
    nw = info.num_cores * info.num_subcores
    unit = nw * C
    NPAD = pl.cdiv(N, unit) * unit
    nb = edge_inx_list[:, 1, :].reshape(V, N, STRIDE).astype(jnp.int32)
    nbT = jnp.transpose(nb, (2, 0, 1))
    idx_flat = jnp.pad(nbT, ((0, 0), (0, 0), (0, NPAD - N))).reshape(
        STRIDE, V * NPAD)
    n_chunk = (V * NPAD) // (nw * C)
    idx_chunks = idx_flat.reshape(STRIDE, nw, n_chunk, C).transpose(
        1, 2, 0, 3).reshape(-1)

    kf = jnp.asarray(k, F32)
    wslot = jnp.where(jnp.arange(STRIDE) < k,
                      1.0 / jnp.maximum(kf, 1.0), 0.0).astype(F32)
    wmat = jnp.broadcast_to(wslot[:, None], (STRIDE, 16)).reshape(-1)

    sc = _make_sc_gather(N, NPAD, STRIDE, HID, V)
    nbr = sc(table, idx_chunks, wmat).reshape(V, NPAD, HID)

    gate = (jnp.asarray(update_epoch) > 0).astype(F32) / V
    scale_row = jnp.broadcast_to(gate, (1, HID)).astype(F32)
    grid3 = pl.cdiv(N, BN3)
    out = pl.pallas_call(
        _k3_body,
        grid=(grid3,),
        in_specs=[
            pl.BlockSpec((V, BN3, HID), lambda i: (0, i, 0)),
            pl.BlockSpec((V, BN3, HID), lambda i: (0, i, 0)),
            pl.BlockSpec((1, HID), lambda i: (0, 0)),
        ],
        out_specs=pl.BlockSpec((BN3, HID), lambda i: (i, 0)),
        out_shape=jax.ShapeDtypeStruct((N, HID), F32),
    )(pre, nbr, scale_row)
    return out


# speedup vs baseline: 3.7651x; 1.2754x over previous
import functools

import jax
import jax.numpy as jnp
from jax import lax
from jax.experimental import pallas as pl
from jax.experimental.pallas import tpu as pltpu
from jax.experimental.pallas import tpu_sc as plsc

F32 = jnp.float32

BN1 = 512
BN3 = 512
C = 32


def _k1_body(tv_ref, win_ref, bin_ref, w0_ref, b0_ref, w1_ref, b1_ref,
             w2_ref, b2_ref, w1h_ref, pre_ref):
    V = tv_ref.shape[0]
    tv = tv_ref[...]
    w0 = w0_ref[...]
    w1 = w1_ref[...]
    w2 = w2_ref[...]
    b0 = b0_ref[...]
    b1 = b1_ref[...]
    b2 = b2_ref[...]
    hi = [jnp.dot(tv[x], win_ref[x], preferred_element_type=F32) + bin_ref[x]
          for x in range(V)]
    s = hi[0]
    for x in range(1, V):
        s = s + hi[x]
    inv = 1.0 / (V - 1)
    for x in range(V):
        hb = jnp.dot(hi[x], w0, preferred_element_type=F32) + b0
        w1h_ref[x] = jnp.dot(hb, w1, preferred_element_type=F32) + b1
        pre_ref[x] = hb + jnp.dot(s - hi[x], w2, preferred_element_type=F32) * inv + b2


def _k3_body(pre_ref, nbr_ref, scale_ref, out_ref):
    V = pre_ref.shape[0]
    acc = jnp.maximum(pre_ref[0] + nbr_ref[0], 0.0)
    for x in range(1, V):
        acc = acc + jnp.maximum(pre_ref[x] + nbr_ref[x], 0.0)
    out_ref[...] = acc * scale_ref[...]


def _make_sc_gather(n_rows, n_pad_per_view, stride, hid, n_view):
    info = plsc.get_sparse_core_info()
    nc, ns, nl = info.num_cores, info.num_subcores, info.num_lanes
    nw = nc * ns
    nt = n_view * n_pad_per_view
    per_w = nt // nw
    n_chunk = per_w // C
    assert per_w % C == 0 and nt % nw == 0
    lane_groups = hid // nl

    assert n_chunk % 2 == 0
    mesh = plsc.VectorSubcoreMesh(core_axis_name="c", subcore_axis_name="s")

    @functools.partial(
        pl.kernel,
        mesh=mesh,
        out_type=jax.ShapeDtypeStruct((nt, hid), F32),
        scratch_types=[
            pltpu.VMEM((stride * C,), jnp.int32),
            pltpu.VMEM((stride * C,), jnp.int32),
            pltpu.VMEM((stride * C, hid), F32),
            pltpu.VMEM((stride * C, hid), F32),
            pltpu.VMEM((C, hid), F32),
            pltpu.VMEM((C, hid), F32),
            pltpu.VMEM((stride * nl,), F32),
            pltpu.SemaphoreType.DMA,
            pltpu.SemaphoreType.DMA,
            pltpu.SemaphoreType.DMA,
            pltpu.SemaphoreType.DMA,
            pltpu.SemaphoreType.DMA,
            pltpu.SemaphoreType.DMA,
        ],
    )
    def sc_gather(table_hbm, idx_hbm, w_hbm, nbr_hbm,
                  idx_v0, idx_v1, g_v0, g_v1, out_v0, out_v1, w_v,
                  sem_i0, sem_i1, sem_g0, sem_g1, sem_o0, sem_o1):
        idx_b = (idx_v0, idx_v1)
        g_b = (g_v0, g_v1)
        out_b = (out_v0, out_v1)
        sem_i = (sem_i0, sem_i1)
        sem_g = (sem_g0, sem_g1)
        sem_o = (sem_o0, sem_o1)
        wid = lax.axis_index("s") * nc + lax.axis_index("c")
        base0 = wid * per_w
        pltpu.sync_copy(w_hbm, w_v)
        wvecs = [w_v[pl.ds(j * nl, nl)] for j in range(stride)]

        def idx_copy(ci, b):
            return pltpu.make_async_copy(
                idx_hbm.at[pl.ds((base0 + ci * C) * stride, stride * C)],
                idx_b[b], sem_i[b])

        n_gath = 2
        rows_per_gath = stride * C // n_gath
        assert stride * C % n_gath == 0 and rows_per_gath <= 128

        def gather_copy(b, j):
            return pltpu.make_async_copy(
                table_hbm.at[idx_b[b].at[pl.ds(j * rows_per_gath,
                                               rows_per_gath)]],
                g_b[b].at[pl.ds(j * rows_per_gath, rows_per_gath)],
                sem_g[b])

        def out_copy(ci, b):
            return pltpu.make_async_copy(
                out_b[b], nbr_hbm.at[pl.ds(base0 + ci * C, C)], sem_o[b])

        def fire_gathers(ci, b):
            off = ((base0 + ci * C) // n_pad_per_view) * n_rows
            for t in range(stride * C // nl):
                sl = pl.ds(t * nl, nl)
                idx_b[b][sl] = idx_b[b][sl] + off
            for j in range(n_gath):
                gather_copy(b, j).start()

        idx_copy(0, 0).start()
        idx_copy(0, 0).wait()
        fire_gathers(0, 0)
        idx_copy(1, 1).start()

        def chunk_step(ci, b):
            for j in range(n_gath):
                gather_copy(b, j).wait()

            @pl.when(ci + 1 < n_chunk)
            def _fire_next():
                idx_copy(ci + 1, 1 - b).wait()
                fire_gathers(ci + 1, 1 - b)

            @pl.when(ci + 2 < n_chunk)
            def _prefetch_idx():
                idx_copy(ci + 2, b).start()

            @pl.when(ci >= 2)
            def _drain_out():
                out_copy(ci - 2, b).wait()

            def row_body(r, carry2):
                for li in range(lane_groups):
                    sl = pl.ds(li * nl, nl)
                    acc = g_b[b][r, sl] * wvecs[0]
                    for j in range(1, stride):
                        acc = acc + g_b[b][j * C + r, sl] * wvecs[j]
                    out_b[b][r, sl] = acc
                return carry2

            lax.fori_loop(0, C, row_body, 0)
            out_copy(ci, b).start()

        def group_body(gi, carry):
            for b in range(2):
                chunk_step(gi * 2 + b, b)
            return carry

        lax.fori_loop(0, n_chunk // 2, group_body, 0)
        out_copy(n_chunk - 2, 0).wait()
        out_copy(n_chunk - 1, 1).wait()

    return sc_gather


def kernel(train_view, k, edge_inx_list, update_epoch, alpha_list, Win, b_in,
           W0, b0, W1, b1, W2, b2):
    V, N, D = train_view.shape
    HID = W0.shape[0]
    STRIDE = edge_inx_list.shape[2] // N

    grid1 = pl.cdiv(N, BN1)
    w1h, pre = pl.pallas_call(
        _k1_body,
        grid=(grid1,),
        in_specs=[
            pl.BlockSpec((V, BN1, D), lambda i: (0, i, 0)),
            pl.BlockSpec((V, D, HID), lambda i: (0, 0, 0)),
            pl.BlockSpec((V, 1, HID), lambda i: (0, 0, 0)),
            pl.BlockSpec((HID, HID), lambda i: (0, 0)),
            pl.BlockSpec((1, HID), lambda i: (0, 0)),
            pl.BlockSpec((HID, HID), lambda i: (0, 0)),
            pl.BlockSpec((1, HID), lambda i: (0, 0)),
            pl.BlockSpec((HID, HID), lambda i: (0, 0)),
            pl.BlockSpec((1, HID), lambda i: (0, 0)),
        ],
        out_specs=[
            pl.BlockSpec((V, BN1, HID), lambda i: (0, i, 0)),
            pl.BlockSpec((V, BN1, HID), lambda i: (0, i, 0)),
        ],
        out_shape=[
            jax.ShapeDtypeStruct((V, N, HID), F32),
            jax.ShapeDtypeStruct((V, N, HID), F32),
        ],
    )(train_view, Win, b_in.reshape(V, 1, HID), W0, b0.reshape(1, HID),
      W1, b1.reshape(1, HID), W2, b2.reshape(1, HID))

    table = w1h.reshape(V * N, HID)

    info = plsc.get_sparse_core_info()
    nw = info.num_cores * info.num_subcores
    unit = nw * C
    NPAD = pl.cdiv(N, unit) * unit
    nb = edge_inx_list[:, 1, :].reshape(V, N, STRIDE).astype(jnp.int32)
    nbT = jnp.transpose(nb, (2, 0, 1))
    idx_flat = jnp.pad(nbT, ((0, 0), (0, 0), (0, NPAD - N))).reshape(
        STRIDE, V * NPAD)
    n_chunk = (V * NPAD) // (nw * C)
    idx_chunks = idx_flat.reshape(STRIDE, nw, n_chunk, C).transpose(
        1, 2, 0, 3).reshape(-1)

    kf = jnp.asarray(k, F32)
    wslot = jnp.where(jnp.arange(STRIDE) < k,
                      1.0 / jnp.maximum(kf, 1.0), 0.0).astype(F32)
    wmat = jnp.broadcast_to(wslot[:, None], (STRIDE, 16)).reshape(-1)

    sc = _make_sc_gather(N, NPAD, STRIDE, HID, V)
    nbr = sc(table, idx_chunks, wmat).reshape(V, NPAD, HID)

    gate = (jnp.asarray(update_epoch) > 0).astype(F32) / V
    scale_row = jnp.broadcast_to(gate, (1, HID)).astype(F32)
    grid3 = pl.cdiv(N, BN3)
    out = pl.pallas_call(
        _k3_body,
        grid=(grid3,),
        in_specs=[
            pl.BlockSpec((V, BN3, HID), lambda i: (0, i, 0)),
            pl.BlockSpec((V, BN3, HID), lambda i: (0, i, 0)),
            pl.BlockSpec((1, HID), lambda i: (0, 0)),
        ],
        out_specs=pl.BlockSpec((BN3, HID), lambda i: (i, 0)),
        out_shape=jax.ShapeDtypeStruct((N, HID), F32),
    )(pre, nbr, scale_row)
    return out


# speedup vs baseline: 3.7680x; 1.0008x over previous
import functools

import jax
import jax.numpy as jnp
from jax import lax
from jax.experimental import pallas as pl
from jax.experimental.pallas import tpu as pltpu
from jax.experimental.pallas import tpu_sc as plsc

F32 = jnp.float32

BN1 = 512
BN3 = 512
C = 32


def _k1_body(tv_ref, win_ref, bin_ref, w0_ref, b0_ref, w1_ref, b1_ref,
             w2_ref, b2_ref, w1h_ref, pre_ref):
    V = tv_ref.shape[0]
    tv = tv_ref[...]
    w0 = w0_ref[...]
    w1 = w1_ref[...]
    w2 = w2_ref[...]
    b0 = b0_ref[...]
    b1 = b1_ref[...]
    b2 = b2_ref[...]
    bf = jnp.bfloat16
    w0b, w1b, w2b = w0.astype(bf), w1.astype(bf), w2.astype(bf)
    hi = [jnp.dot(tv[x].astype(bf), win_ref[x].astype(bf),
                  preferred_element_type=F32) + bin_ref[x]
          for x in range(V)]
    s = hi[0]
    for x in range(1, V):
        s = s + hi[x]
    inv = 1.0 / (V - 1)
    for x in range(V):
        hb = jnp.dot(hi[x].astype(bf), w0b, preferred_element_type=F32) + b0
        w1h_ref[x] = jnp.dot(hb.astype(bf), w1b, preferred_element_type=F32) + b1
        pre_ref[x] = hb + jnp.dot((s - hi[x]).astype(bf), w2b,
                                  preferred_element_type=F32) * inv + b2


def _k3_body(pre_ref, nbr_ref, scale_ref, out_ref):
    V = pre_ref.shape[0]
    acc = jnp.maximum(pre_ref[0] + nbr_ref[0], 0.0)
    for x in range(1, V):
        acc = acc + jnp.maximum(pre_ref[x] + nbr_ref[x], 0.0)
    out_ref[...] = acc * scale_ref[...]


def _make_sc_gather(n_rows, n_pad_per_view, stride, hid, n_view):
    info = plsc.get_sparse_core_info()
    nc, ns, nl = info.num_cores, info.num_subcores, info.num_lanes
    nw = nc * ns
    nt = n_view * n_pad_per_view
    per_w = nt // nw
    n_chunk = per_w // C
    assert per_w % C == 0 and nt % nw == 0
    lane_groups = hid // nl

    assert n_chunk % 2 == 0
    mesh = plsc.VectorSubcoreMesh(core_axis_name="c", subcore_axis_name="s")

    @functools.partial(
        pl.kernel,
        mesh=mesh,
        out_type=jax.ShapeDtypeStruct((nt, hid), F32),
        scratch_types=[
            pltpu.VMEM((stride * C,), jnp.int32),
            pltpu.VMEM((stride * C,), jnp.int32),
            pltpu.VMEM((stride * C, hid), F32),
            pltpu.VMEM((stride * C, hid), F32),
            pltpu.VMEM((C, hid), F32),
            pltpu.VMEM((C, hid), F32),
            pltpu.VMEM((stride * nl,), F32),
            pltpu.SemaphoreType.DMA,
            pltpu.SemaphoreType.DMA,
            pltpu.SemaphoreType.DMA,
            pltpu.SemaphoreType.DMA,
            pltpu.SemaphoreType.DMA,
            pltpu.SemaphoreType.DMA,
        ],
    )
    def sc_gather(table_hbm, idx_hbm, w_hbm, nbr_hbm,
                  idx_v0, idx_v1, g_v0, g_v1, out_v0, out_v1, w_v,
                  sem_i0, sem_i1, sem_g0, sem_g1, sem_o0, sem_o1):
        idx_b = (idx_v0, idx_v1)
        g_b = (g_v0, g_v1)
        out_b = (out_v0, out_v1)
        sem_i = (sem_i0, sem_i1)
        sem_g = (sem_g0, sem_g1)
        sem_o = (sem_o0, sem_o1)
        wid = lax.axis_index("s") * nc + lax.axis_index("c")
        base0 = wid * per_w
        pltpu.sync_copy(w_hbm, w_v)
        wvecs = [w_v[pl.ds(j * nl, nl)] for j in range(stride)]

        def idx_copy(ci, b):
            return pltpu.make_async_copy(
                idx_hbm.at[pl.ds((base0 + ci * C) * stride, stride * C)],
                idx_b[b], sem_i[b])

        n_gath = 2
        rows_per_gath = stride * C // n_gath
        assert stride * C % n_gath == 0 and rows_per_gath <= 128

        def gather_copy(b, j):
            return pltpu.make_async_copy(
                table_hbm.at[idx_b[b].at[pl.ds(j * rows_per_gath,
                                               rows_per_gath)]],
                g_b[b].at[pl.ds(j * rows_per_gath, rows_per_gath)],
                sem_g[b])

        def out_copy(ci, b):
            return pltpu.make_async_copy(
                out_b[b], nbr_hbm.at[pl.ds(base0 + ci * C, C)], sem_o[b])

        def fire_gathers(ci, b):
            off = ((base0 + ci * C) // n_pad_per_view) * n_rows
            for t in range(stride * C // nl):
                sl = pl.ds(t * nl, nl)
                idx_b[b][sl] = idx_b[b][sl] + off
            for j in range(n_gath):
                gather_copy(b, j).start()

        idx_copy(0, 0).start()
        idx_copy(0, 0).wait()
        fire_gathers(0, 0)
        idx_copy(1, 1).start()

        def chunk_step(ci, b):
            for j in range(n_gath):
                gather_copy(b, j).wait()

            @pl.when(ci + 1 < n_chunk)
            def _fire_next():
                idx_copy(ci + 1, 1 - b).wait()
                fire_gathers(ci + 1, 1 - b)

            @pl.when(ci + 2 < n_chunk)
            def _prefetch_idx():
                idx_copy(ci + 2, b).start()

            @pl.when(ci >= 2)
            def _drain_out():
                out_copy(ci - 2, b).wait()

            def row_body(r, carry2):
                for li in range(lane_groups):
                    sl = pl.ds(li * nl, nl)
                    acc = g_b[b][r, sl] * wvecs[0]
                    for j in range(1, stride):
                        acc = acc + g_b[b][j * C + r, sl] * wvecs[j]
                    out_b[b][r, sl] = acc
                return carry2

            lax.fori_loop(0, C, row_body, 0)
            out_copy(ci, b).start()

        def group_body(gi, carry):
            for b in range(2):
                chunk_step(gi * 2 + b, b)
            return carry

        lax.fori_loop(0, n_chunk // 2, group_body, 0)
        out_copy(n_chunk - 2, 0).wait()
        out_copy(n_chunk - 1, 1).wait()

    return sc_gather


def kernel(train_view, k, edge_inx_list, update_epoch, alpha_list, Win, b_in,
           W0, b0, W1, b1, W2, b2):
    V, N, D = train_view.shape
    HID = W0.shape[0]
    STRIDE = edge_inx_list.shape[2] // N

    grid1 = pl.cdiv(N, BN1)
    w1h, pre = pl.pallas_call(
        _k1_body,
        grid=(grid1,),
        in_specs=[
            pl.BlockSpec((V, BN1, D), lambda i: (0, i, 0)),
            pl.BlockSpec((V, D, HID), lambda i: (0, 0, 0)),
            pl.BlockSpec((V, 1, HID), lambda i: (0, 0, 0)),
            pl.BlockSpec((HID, HID), lambda i: (0, 0)),
            pl.BlockSpec((1, HID), lambda i: (0, 0)),
            pl.BlockSpec((HID, HID), lambda i: (0, 0)),
            pl.BlockSpec((1, HID), lambda i: (0, 0)),
            pl.BlockSpec((HID, HID), lambda i: (0, 0)),
            pl.BlockSpec((1, HID), lambda i: (0, 0)),
        ],
        out_specs=[
            pl.BlockSpec((V, BN1, HID), lambda i: (0, i, 0)),
            pl.BlockSpec((V, BN1, HID), lambda i: (0, i, 0)),
        ],
        out_shape=[
            jax.ShapeDtypeStruct((V, N, HID), F32),
            jax.ShapeDtypeStruct((V, N, HID), F32),
        ],
    )(train_view, Win, b_in.reshape(V, 1, HID), W0, b0.reshape(1, HID),
      W1, b1.reshape(1, HID), W2, b2.reshape(1, HID))

    table = w1h.reshape(V * N, HID)

    info = plsc.get_sparse_core_info()
    nw = info.num_cores * info.num_subcores
    unit = nw * C
    NPAD = pl.cdiv(N, unit) * unit
    nb = edge_inx_list[:, 1, :].reshape(V, N, STRIDE).astype(jnp.int32)
    nbT = jnp.transpose(nb, (2, 0, 1))
    idx_flat = jnp.pad(nbT, ((0, 0), (0, 0), (0, NPAD - N))).reshape(
        STRIDE, V * NPAD)
    n_chunk = (V * NPAD) // (nw * C)
    idx_chunks = idx_flat.reshape(STRIDE, nw, n_chunk, C).transpose(
        1, 2, 0, 3).reshape(-1)

    kf = jnp.asarray(k, F32)
    wslot = jnp.where(jnp.arange(STRIDE) < k,
                      1.0 / jnp.maximum(kf, 1.0), 0.0).astype(F32)
    wmat = jnp.broadcast_to(wslot[:, None], (STRIDE, 16)).reshape(-1)

    sc = _make_sc_gather(N, NPAD, STRIDE, HID, V)
    nbr = sc(table, idx_chunks, wmat).reshape(V, NPAD, HID)

    gate = (jnp.asarray(update_epoch) > 0).astype(F32) / V
    scale_row = jnp.broadcast_to(gate, (1, HID)).astype(F32)
    grid3 = pl.cdiv(N, BN3)
    out = pl.pallas_call(
        _k3_body,
        grid=(grid3,),
        in_specs=[
            pl.BlockSpec((V, BN3, HID), lambda i: (0, i, 0)),
            pl.BlockSpec((V, BN3, HID), lambda i: (0, i, 0)),
            pl.BlockSpec((1, HID), lambda i: (0, 0)),
        ],
        out_specs=pl.BlockSpec((BN3, HID), lambda i: (i, 0)),
        out_shape=jax.ShapeDtypeStruct((N, HID), F32),
    )(pre, nbr, scale_row)
    return out
